# Initial kernel scaffold; baseline (speedup 1.0000x reference)
import jax
import jax.numpy as jnp
from jax.experimental import pallas as pl


def kernel(h, r, t, w, n_hn, n_rel_hn, n_t, n_h, n_rel_tn, n_tn, s_h, s_r, s_t, s_w, W_eh, W_et, W_r, W_ri, lin_w, lin_b):
    raise NotImplementedError("write your pallas kernel here")


# speedup vs baseline: 1.2631x; 1.2631x over previous
import functools

import jax
import jax.numpy as jnp
from jax import lax
from jax.experimental import pallas as pl
from jax.experimental.pallas import tpu as pltpu
from jax.experimental.pallas import tpu_sc as plsc

_NUM_CONS = 1000000
_NUM_RELS = 1000
_DIM = 32
_B = 4096
_NEG = 10
_REG_SCALE = 0.01
_P_NEG = 1.0

_L = 16
_NW = 32
_K = 128
_POS_PER_W = _B // _NW
_NNEG = 2 * _B * _NEG
_NEG_PER_W = _NNEG // _NW
_NCHUNK_NEG = _NEG_PER_W // _K


def _sc_body(ph, pr, pt, pw, nh, nr, nt, pa, pb,
             W_eh, W_et, W_r, W_ri, out,
             ih0, ir0, it0, ih1, ir1, it1,
             Eh0, Et0, Ft0, Fh0, R0, Ri0,
             Eh1, Et1, Ft1, Fh1, R1, Ri1,
             scores, wv, va, vb, accP, accN, accR, sem0, sem1):
    cid = lax.axis_index("c")
    sid = lax.axis_index("s")
    wid = sid * 2 + cid

    idx_slots = ((ih0, ir0, it0), (ih1, ir1, it1))
    row_slots = ((Eh0, Et0, Ft0, Fh0, R0, Ri0),
                 (Eh1, Et1, Ft1, Fh1, R1, Ri1))
    sems = (sem0, sem1)

    zeros = jnp.zeros((_L,), jnp.float32)
    accP[...] = zeros
    accN[...] = zeros
    accR[...] = zeros
    pltpu.sync_copy(pa, va)
    pltpu.sync_copy(pb, vb)
    a_vec = va[...]
    b_vec = vb[...]
    iota16 = lax.iota(jnp.int32, _L)

    def issue(slot, h_hbm, r_hbm, t_hbm, start):
        ih, ir, it = idx_slots[slot]
        pltpu.sync_copy(h_hbm.at[pl.ds(start, _K)], ih)
        pltpu.sync_copy(r_hbm.at[pl.ds(start, _K)], ir)
        pltpu.sync_copy(t_hbm.at[pl.ds(start, _K)], it)
        Eh, Et, Ft, Fh, R, Ri = row_slots[slot]
        sem = sems[slot]
        pltpu.async_copy(W_eh.at[ih], Eh, sem)
        pltpu.async_copy(W_eh.at[it], Et, sem)
        pltpu.async_copy(W_et.at[it], Ft, sem)
        pltpu.async_copy(W_et.at[ih], Fh, sem)
        pltpu.async_copy(W_r.at[ir], R, sem)
        pltpu.async_copy(W_ri.at[ir], Ri, sem)

    def drain(slot):
        Eh, Et, Ft, Fh, R, Ri = row_slots[slot]
        ih, ir, it = idx_slots[slot]
        sem = sems[slot]
        pltpu.make_async_copy(W_eh.at[ih], Eh, sem).wait()
        pltpu.make_async_copy(W_eh.at[it], Et, sem).wait()
        pltpu.make_async_copy(W_et.at[it], Ft, sem).wait()
        pltpu.make_async_copy(W_et.at[ih], Fh, sem).wait()
        pltpu.make_async_copy(W_r.at[ir], R, sem).wait()
        pltpu.make_async_copy(W_ri.at[ir], Ri, sem).wait()

    def compute(slot, is_pos):
        Eh, Et, Ft, Fh, R, Ri = row_slots[slot]

        def tri_body(i, carry):
            a0 = Eh[i, pl.ds(0, _L)]
            a1 = Eh[i, pl.ds(_L, _L)]
            b0 = Et[i, pl.ds(0, _L)]
            b1 = Et[i, pl.ds(_L, _L)]
            c0 = Ft[i, pl.ds(0, _L)]
            c1 = Ft[i, pl.ds(_L, _L)]
            d0 = Fh[i, pl.ds(0, _L)]
            d1 = Fh[i, pl.ds(_L, _L)]
            r0 = R[i, pl.ds(0, _L)]
            r1 = R[i, pl.ds(_L, _L)]
            q0 = Ri[i, pl.ds(0, _L)]
            q1 = Ri[i, pl.ds(_L, _L)]
            v = (a0 * r0 * c0 + a1 * r1 * c1) + (b0 * q0 * d0 + b1 * q1 * d1)
            scores[i, pl.ds(0, _L)] = v
            if is_pos:
                sq = ((a0 * a0 + a1 * a1) + (b0 * b0 + b1 * b1)
                      + (c0 * c0 + c1 * c1) + (d0 * d0 + d1 * d1)
                      + (r0 * r0 + r1 * r1) + (q0 * q0 + q1 * q1))
                accR[...] = accR[...] + sq
            return carry

        lax.fori_loop(0, _K, tri_body, 0)

        def grp_body(g, carry):
            rows = iota16 + g * _L
            s = plsc.load_gather(scores, [rows, jnp.zeros((_L,), jnp.int32)])
            for k in range(1, _L):
                s = s + plsc.load_gather(
                    scores, [rows, jnp.full((_L,), k, jnp.int32)])
            x = s * a_vec + b_vec
            p = 1.0 / (1.0 + jnp.exp(-x))
            if is_pos:
                d = p - wv[pl.ds(g * _L, _L)]
                accP[...] = accP[...] + d * d
            else:
                accN[...] = accN[...] + p * p
            return carry

        lax.fori_loop(0, _K // _L, grp_body, 0)

    pstart = wid * _POS_PER_W
    issue(0, ph, pr, pt, pstart)
    pltpu.sync_copy(pw.at[pl.ds(pstart, _K)], wv)
    drain(0)
    compute(0, True)

    nbase = wid * _NEG_PER_W
    issue(0, nh, nr, nt, nbase)

    def pair_body(g2, carry):
        c0 = g2 * 2
        issue(1, nh, nr, nt, nbase + (c0 + 1) * _K)
        drain(0)
        compute(0, False)

        @pl.when(c0 + 2 < _NCHUNK_NEG)
        def _():
            issue(0, nh, nr, nt, nbase + (c0 + 2) * _K)

        drain(1)
        compute(1, False)
        return carry

    lax.fori_loop(0, _NCHUNK_NEG // 2, pair_body, 0)

    pltpu.sync_copy(accP, out.at[0, wid])
    pltpu.sync_copy(accN, out.at[1, wid])
    pltpu.sync_copy(accR, out.at[2, wid])


def _make_sc_call():
    mesh = plsc.VectorSubcoreMesh(core_axis_name="c", subcore_axis_name="s")
    idx_t = [pltpu.VMEM((_K,), jnp.int32) for _ in range(6)]
    row_t = [pltpu.VMEM((_K, _DIM), jnp.float32) for _ in range(12)]
    return pl.kernel(
        _sc_body,
        out_type=jax.ShapeDtypeStruct((3, _NW, _L), jnp.float32),
        mesh=mesh,
        compiler_params=pltpu.CompilerParams(
            needs_layout_passes=False, use_tc_tiling_on_sc=False),
        scratch_types=idx_t + row_t + [
            pltpu.VMEM((_K, _L), jnp.float32),
            pltpu.VMEM((_K,), jnp.float32),
            pltpu.VMEM((_L,), jnp.float32),
            pltpu.VMEM((_L,), jnp.float32),
            pltpu.VMEM((_L,), jnp.float32),
            pltpu.VMEM((_L,), jnp.float32),
            pltpu.VMEM((_L,), jnp.float32),
            pltpu.SemaphoreType.DMA,
            pltpu.SemaphoreType.DMA,
        ],
    )


_sc_call = _make_sc_call()


def kernel(h, r, t, w, n_hn, n_rel_hn, n_t, n_h, n_rel_tn, n_tn,
           s_h, s_r, s_t, s_w, W_eh, W_et, W_r, W_ri, lin_w, lin_b):
    ph = h.astype(jnp.int32)
    pr = r.astype(jnp.int32)
    pt = t.astype(jnp.int32)
    nh = jnp.concatenate([n_hn.reshape(-1), n_h.reshape(-1)]).astype(jnp.int32)
    nr = jnp.concatenate([n_rel_hn.reshape(-1), n_rel_tn.reshape(-1)]).astype(jnp.int32)
    nt = jnp.concatenate([n_t.reshape(-1), n_tn.reshape(-1)]).astype(jnp.int32)
    pa = jnp.broadcast_to(0.5 * lin_w[0, 0], (_L,)).astype(jnp.float32)
    pb = jnp.broadcast_to(lin_b[0], (_L,)).astype(jnp.float32)
    w32 = w.astype(jnp.float32)

    out = _sc_call(ph, pr, pt, w32, nh, nr, nt, pa, pb,
                   W_eh, W_et, W_r, W_ri)

    pos_sum = jnp.sum(out[0])
    neg_sum = jnp.sum(out[1])
    reg_sum = jnp.sum(out[2])
    this_loss = (pos_sum + neg_sum * (_P_NEG / (2.0 * _NEG))) / _B
    return this_loss + _REG_SCALE * reg_sum / (2.0 * _B)
